# baseline (speedup 1.0000x reference)
import functools

import jax
import jax.numpy as jnp
from jax import lax
from jax.experimental import pallas as pl
from jax.experimental.pallas import tpu as pltpu
from jax.experimental.pallas import tpu_sc as plsc

N_NODES, D_N = 10000, 128
N_EDGES, D_E = 320000, 16
G = 64
NC, NS = 2, 16
NW = NC * NS
LANES = 16
NJ = D_N // LANES
NODE_CHUNK = 304
NODE_TAIL = N_NODES - NW * NODE_CHUNK
NODE_TAIL_GROUPS = NODE_TAIL // LANES
EDGE_PER_W = N_EDGES // NW
E_CHUNK = 2000
N_ECHUNKS = EDGE_PER_W // E_CHUNK


def _sc_node_partials(node_feat, node_ids):
    mesh = plsc.VectorSubcoreMesh(core_axis_name="c", subcore_axis_name="s")

    @functools.partial(
        pl.kernel,
        out_type=jax.ShapeDtypeStruct((NW, G, D_N), jnp.float32),
        mesh=mesh,
        compiler_params=pltpu.CompilerParams(use_tc_tiling_on_sc=False,
                                             needs_layout_passes=False),
        scratch_types=[
            pltpu.VMEM((NODE_CHUNK, D_N), jnp.float32),
            pltpu.VMEM((NODE_CHUNK,), jnp.int32),
            pltpu.VMEM((LANES, D_N), jnp.float32),
            pltpu.VMEM((LANES,), jnp.int32),
            pltpu.VMEM((G, D_N), jnp.float32),
            pltpu.SemaphoreType.DMA,
            pltpu.SemaphoreType.DMA,
        ],
    )
    def k(nf_hbm, nid_hbm, pn_hbm, nbuf, nidv, ntbuf, ntidv, acc_n,
          sem_n, sem_ni):
        wid = lax.axis_index("c") * NS + lax.axis_index("s")
        zero = jnp.zeros((LANES,), jnp.float32)

        nbase = wid * NODE_CHUNK
        h_ni = pltpu.async_copy(nid_hbm.at[pl.ds(nbase, NODE_CHUNK)], nidv,
                                sem_ni)
        h_n = pltpu.async_copy(nf_hbm.at[pl.ds(nbase, NODE_CHUNK)], nbuf,
                               sem_n)

        def zbody(g, carry):
            for j in range(NJ):
                acc_n[g, pl.ds(j * LANES, LANES)] = zero
            return carry
        lax.fori_loop(0, G, zbody, 0)

        h_ni.wait()
        h_n.wait()

        def nbody(grp, carry):
            prev, run = carry[0], carry[1:]
            i0 = grp * LANES
            gids = nidv[pl.ds(i0, LANES)]
            last = gids[LANES - 1]
            rows = [[nbuf[i0 + l, pl.ds(j * LANES, LANES)] for j in range(NJ)]
                    for l in range(LANES)]

            def same_fn():
                s = list(run)
                for l in range(LANES):
                    s = [s[j] + rows[l][j] for j in range(NJ)]
                return (prev,) + tuple(s)

            def diff_fn():
                for j in range(NJ):
                    plsc.addupdate(acc_n.at[prev, pl.ds(j * LANES, LANES)],
                                   run[j])
                for l in range(LANES):
                    g = gids[l]
                    for j in range(NJ):
                        plsc.addupdate(acc_n.at[g, pl.ds(j * LANES, LANES)],
                                       rows[l][j])
                return (last,) + (zero,) * NJ

            return lax.cond(last == prev, same_fn, diff_fn)

        ncarry = lax.fori_loop(0, NODE_CHUNK // LANES, nbody,
                               (jnp.int32(0),) + (zero,) * NJ)
        for j in range(NJ):
            plsc.addupdate(acc_n.at[ncarry[0], pl.ds(j * LANES, LANES)],
                           ncarry[1 + j])

        @pl.when(wid < NODE_TAIL_GROUPS)
        def _tail():
            tb = NW * NODE_CHUNK + wid * LANES
            pltpu.sync_copy(nid_hbm.at[pl.ds(tb, LANES)], ntidv)
            pltpu.sync_copy(nf_hbm.at[pl.ds(tb, LANES)], ntbuf)
            gids = ntidv[...]
            for l in range(LANES):
                g = gids[l]
                for j in range(NJ):
                    plsc.addupdate(acc_n.at[g, pl.ds(j * LANES, LANES)],
                                   ntbuf[l, pl.ds(j * LANES, LANES)])

        pltpu.sync_copy(acc_n, pn_hbm.at[wid])

    return k(node_feat, node_ids)


def _sc_edge_partials(edge_feat_t, edge_ids):
    mesh = plsc.VectorSubcoreMesh(core_axis_name="c", subcore_axis_name="s")

    @functools.partial(
        pl.kernel,
        out_type=jax.ShapeDtypeStruct((NW, D_E, G), jnp.float32),
        mesh=mesh,
        compiler_params=pltpu.CompilerParams(use_tc_tiling_on_sc=False,
                                             needs_layout_passes=False),
        scratch_types=[
            pltpu.VMEM((D_E, E_CHUNK), jnp.float32),
            pltpu.VMEM((D_E, E_CHUNK), jnp.float32),
            pltpu.VMEM((EDGE_PER_W,), jnp.int32),
            pltpu.VMEM((D_E, G), jnp.float32),
            pltpu.SemaphoreType.DMA,
            pltpu.SemaphoreType.DMA,
            pltpu.SemaphoreType.DMA,
        ],
    )
    def k(eft_hbm, eid_hbm, pet_hbm, ebuf0, ebuf1, eidv, acc_et,
          sem_i, sem_e0, sem_e1):
        wid = lax.axis_index("c") * NS + lax.axis_index("s")
        zero = jnp.zeros((LANES,), jnp.float32)
        ebufs = (ebuf0, ebuf1)
        esems = (sem_e0, sem_e1)

        ebase = wid * EDGE_PER_W
        h_id = pltpu.async_copy(eid_hbm.at[pl.ds(ebase, EDGE_PER_W)], eidv,
                                sem_i)
        eh = [None] * N_ECHUNKS
        eh[0] = pltpu.async_copy(eft_hbm.at[:, pl.ds(ebase, E_CHUNK)],
                                 ebufs[0], sem_e0)

        for f in range(D_E):
            for g4 in range(G // LANES):
                acc_et[f, pl.ds(g4 * LANES, LANES)] = zero

        h_id.wait()
        carry = (jnp.int32(0),) + (zero,) * D_E
        for c in range(N_ECHUNKS):
            if c + 1 < N_ECHUNKS:
                nxt = (c + 1) % 2
                eh[c + 1] = pltpu.async_copy(
                    eft_hbm.at[:, pl.ds(ebase + (c + 1) * E_CHUNK, E_CHUNK)],
                    ebufs[nxt], esems[nxt])
            eh[c].wait()
            buf = ebufs[c % 2]

            def ebody(grp, carry, c=c, buf=buf):
                prev, run = carry[0], carry[1:]
                i0 = grp * LANES
                gids = eidv[pl.ds(c * E_CHUNK + i0, LANES)]
                last = gids[LANES - 1]
                v = [buf[f, pl.ds(i0, LANES)] for f in range(D_E)]

                def same_fn():
                    return (prev,) + tuple(run[f] + v[f] for f in range(D_E))

                def diff_fn():
                    pv = jnp.full((LANES,), prev, jnp.int32)
                    for f in range(D_E):
                        plsc.addupdate_scatter(acc_et.at[f], [pv], run[f])
                    for f in range(D_E):
                        plsc.addupdate_scatter(acc_et.at[f], [gids], v[f])
                    return (last,) + (zero,) * D_E

                return lax.cond(last == prev, same_fn, diff_fn)

            carry = lax.fori_loop(0, E_CHUNK // LANES, ebody, carry)

        pv = jnp.full((LANES,), carry[0], jnp.int32)
        for f in range(D_E):
            plsc.addupdate_scatter(acc_et.at[f], [pv], carry[1 + f])

        pltpu.sync_copy(acc_et, pet_hbm.at[wid])

    return k(edge_feat_t, edge_ids)


def _combine_body(pn_ref, pet_ref, out_ref):
    out_ref[:, :D_N] = jnp.sum(pn_ref[...], axis=0)
    es_t = jnp.sum(pet_ref[...], axis=0)
    out_ref[:, D_N:] = es_t.T


def kernel(node_feat, node_graph_ids, edge_feat, edge_graph_ids, num_graphs):
    del num_graphs
    pn = _sc_node_partials(node_feat, node_graph_ids.astype(jnp.int32))
    pet = _sc_edge_partials(edge_feat.T, edge_graph_ids.astype(jnp.int32))
    return pl.pallas_call(
        _combine_body,
        out_shape=jax.ShapeDtypeStruct((G, D_N + D_E), jnp.float32),
    )(pn, pet)


# speedup vs baseline: 1.1048x; 1.1048x over previous
import functools

import jax
import jax.numpy as jnp
from jax import lax
from jax.experimental import pallas as pl
from jax.experimental.pallas import tpu as pltpu
from jax.experimental.pallas import tpu_sc as plsc

N_NODES, D_N = 10000, 128
N_EDGES, D_E = 320000, 16
G = 64
NC, NS = 2, 16
NW = NC * NS
LANES = 16
NJ = D_N // LANES
NODE_CHUNK = 304
NODE_TAIL = N_NODES - NW * NODE_CHUNK
NODE_TAIL_GROUPS = NODE_TAIL // LANES
EDGE_PER_W = N_EDGES // NW
E_CHUNK = 2000
N_ECHUNKS = EDGE_PER_W // E_CHUNK


def _sc_node_partials(node_feat, node_ids):
    mesh = plsc.VectorSubcoreMesh(core_axis_name="c", subcore_axis_name="s")

    @functools.partial(
        pl.kernel,
        out_type=jax.ShapeDtypeStruct((NW, G, D_N), jnp.float32),
        mesh=mesh,
        compiler_params=pltpu.CompilerParams(use_tc_tiling_on_sc=False,
                                             needs_layout_passes=False),
        scratch_types=[
            pltpu.VMEM((NODE_CHUNK, D_N), jnp.float32),
            pltpu.VMEM((NODE_CHUNK,), jnp.int32),
            pltpu.VMEM((LANES, D_N), jnp.float32),
            pltpu.VMEM((LANES,), jnp.int32),
            pltpu.VMEM((G, D_N), jnp.float32),
            pltpu.SemaphoreType.DMA,
            pltpu.SemaphoreType.DMA,
        ],
    )
    def k(nf_hbm, nid_hbm, pn_hbm, nbuf, nidv, ntbuf, ntidv, acc_n,
          sem_n, sem_ni):
        wid = lax.axis_index("c") * NS + lax.axis_index("s")
        zero = jnp.zeros((LANES,), jnp.float32)

        nbase = wid * NODE_CHUNK
        h_ni = pltpu.async_copy(nid_hbm.at[pl.ds(nbase, NODE_CHUNK)], nidv,
                                sem_ni)
        h_n = pltpu.async_copy(nf_hbm.at[pl.ds(nbase, NODE_CHUNK)], nbuf,
                               sem_n)

        def zbody(g, carry):
            for j in range(NJ):
                acc_n[g, pl.ds(j * LANES, LANES)] = zero
            return carry
        lax.fori_loop(0, G, zbody, 0)

        h_ni.wait()
        h_n.wait()

        def nbody(grp, carry):
            prev, run = carry[0], carry[1:]
            i0 = grp * LANES
            gids = nidv[pl.ds(i0, LANES)]
            last = gids[LANES - 1]
            rows = [[nbuf[i0 + l, pl.ds(j * LANES, LANES)] for j in range(NJ)]
                    for l in range(LANES)]

            def same_fn():
                s = list(run)
                for l in range(LANES):
                    s = [s[j] + rows[l][j] for j in range(NJ)]
                return (prev,) + tuple(s)

            def diff_fn():
                for j in range(NJ):
                    plsc.addupdate(acc_n.at[prev, pl.ds(j * LANES, LANES)],
                                   run[j])
                for l in range(LANES):
                    g = gids[l]
                    for j in range(NJ):
                        plsc.addupdate(acc_n.at[g, pl.ds(j * LANES, LANES)],
                                       rows[l][j])
                return (last,) + (zero,) * NJ

            return lax.cond(last == prev, same_fn, diff_fn)

        ncarry = lax.fori_loop(0, NODE_CHUNK // LANES, nbody,
                               (jnp.int32(0),) + (zero,) * NJ)
        for j in range(NJ):
            plsc.addupdate(acc_n.at[ncarry[0], pl.ds(j * LANES, LANES)],
                           ncarry[1 + j])

        @pl.when(wid < NODE_TAIL_GROUPS)
        def _tail():
            tb = NW * NODE_CHUNK + wid * LANES
            pltpu.sync_copy(nid_hbm.at[pl.ds(tb, LANES)], ntidv)
            pltpu.sync_copy(nf_hbm.at[pl.ds(tb, LANES)], ntbuf)
            gids = ntidv[...]
            for l in range(LANES):
                g = gids[l]
                for j in range(NJ):
                    plsc.addupdate(acc_n.at[g, pl.ds(j * LANES, LANES)],
                                   ntbuf[l, pl.ds(j * LANES, LANES)])

        pltpu.sync_copy(acc_n, pn_hbm.at[wid])

    return k(node_feat, node_ids)


def _sc_edge_partials(edge_feat_t, edge_ids):
    mesh = plsc.VectorSubcoreMesh(core_axis_name="c", subcore_axis_name="s")

    @functools.partial(
        pl.kernel,
        out_type=jax.ShapeDtypeStruct((NW, D_E, G), jnp.float32),
        mesh=mesh,
        compiler_params=pltpu.CompilerParams(use_tc_tiling_on_sc=False,
                                             needs_layout_passes=False),
        scratch_types=[
            pltpu.VMEM((D_E, E_CHUNK), jnp.float32),
            pltpu.VMEM((D_E, E_CHUNK), jnp.float32),
            pltpu.VMEM((EDGE_PER_W,), jnp.int32),
            pltpu.VMEM((D_E, G), jnp.float32),
            pltpu.SemaphoreType.DMA,
            pltpu.SemaphoreType.DMA,
            pltpu.SemaphoreType.DMA,
        ],
    )
    def k(eft_hbm, eid_hbm, pet_hbm, ebuf0, ebuf1, eidv, acc_et,
          sem_i, sem_e0, sem_e1):
        wid = lax.axis_index("c") * NS + lax.axis_index("s")
        zero = jnp.zeros((LANES,), jnp.float32)
        ebufs = (ebuf0, ebuf1)
        esems = (sem_e0, sem_e1)

        ebase = wid * EDGE_PER_W
        h_id = pltpu.async_copy(eid_hbm.at[pl.ds(ebase, EDGE_PER_W)], eidv,
                                sem_i)
        eh = [None] * N_ECHUNKS
        eh[0] = pltpu.async_copy(eft_hbm.at[:, pl.ds(ebase, E_CHUNK)],
                                 ebufs[0], sem_e0)

        for f in range(D_E):
            for g4 in range(G // LANES):
                acc_et[f, pl.ds(g4 * LANES, LANES)] = zero

        h_id.wait()
        carry = (jnp.int32(0),) + (zero,) * D_E
        for c in range(N_ECHUNKS):
            if c + 1 < N_ECHUNKS:
                nxt = (c + 1) % 2
                eh[c + 1] = pltpu.async_copy(
                    eft_hbm.at[:, pl.ds(ebase + (c + 1) * E_CHUNK, E_CHUNK)],
                    ebufs[nxt], esems[nxt])
            eh[c].wait()
            buf = ebufs[c % 2]

            def ebody(grp, carry, c=c, buf=buf):
                prev, run = carry[0], carry[1:]
                i0 = grp * LANES
                gids = eidv[pl.ds(c * E_CHUNK + i0, LANES)]
                last = gids[LANES - 1]
                v = [buf[f, pl.ds(i0, LANES)] for f in range(D_E)]

                def same_fn():
                    return (prev,) + tuple(run[f] + v[f] for f in range(D_E))

                def diff_fn():
                    pv = jnp.full((LANES,), prev, jnp.int32)
                    for f in range(D_E):
                        plsc.addupdate_scatter(acc_et.at[f], [pv], run[f])
                    for f in range(D_E):
                        plsc.addupdate_scatter(acc_et.at[f], [gids], v[f])
                    return (last,) + (zero,) * D_E

                return lax.cond(last == prev, same_fn, diff_fn)

            carry = lax.fori_loop(0, E_CHUNK // LANES, ebody, carry)

        pv = jnp.full((LANES,), carry[0], jnp.int32)
        for f in range(D_E):
            plsc.addupdate_scatter(acc_et.at[f], [pv], carry[1 + f])

        pltpu.sync_copy(acc_et, pet_hbm.at[wid])

    return k(edge_feat_t, edge_ids)


def _combine_body(pn_ref, pet_ref, out_ref):
    out_ref[:, :D_N] = jnp.sum(pn_ref[...], axis=0)
    es_t = jnp.sum(pet_ref[...], axis=0)
    out_ref[:, D_N:] = es_t.T


def kernel(node_feat, node_graph_ids, edge_feat, edge_graph_ids, num_graphs):
    del num_graphs
    pn = _sc_node_partials(node_feat, node_graph_ids.astype(jnp.int32))
    pn, eids = lax.optimization_barrier(
        (pn, edge_graph_ids.astype(jnp.int32)))
    pet = _sc_edge_partials(edge_feat.T, eids)
    return pl.pallas_call(
        _combine_body,
        out_shape=jax.ShapeDtypeStruct((G, D_N + D_E), jnp.float32),
    )(pn, pet)


# speedup vs baseline: 1.1568x; 1.0471x over previous
import functools

import jax
import jax.numpy as jnp
from jax import lax
from jax.experimental import pallas as pl
from jax.experimental.pallas import tpu as pltpu
from jax.experimental.pallas import tpu_sc as plsc

N_NODES, D_N = 10000, 128
N_EDGES, D_E = 320000, 16
G = 64
NC, NS = 2, 16
NW = NC * NS
LANES = 16
NJ = D_N // LANES
NODE_CHUNK = 304
NODE_TAIL = N_NODES - NW * NODE_CHUNK
NODE_TAIL_GROUPS = NODE_TAIL // LANES
EDGE_PER_W = N_EDGES // NW
E_CHUNK = 2000
N_ECHUNKS = EDGE_PER_W // E_CHUNK
E_SUBS = [512, 512, 512, 464]


def _sc_node_partials(node_feat, node_ids):
    mesh = plsc.VectorSubcoreMesh(core_axis_name="c", subcore_axis_name="s")

    @functools.partial(
        pl.kernel,
        out_type=jax.ShapeDtypeStruct((NW, G, D_N), jnp.float32),
        mesh=mesh,
        compiler_params=pltpu.CompilerParams(use_tc_tiling_on_sc=False,
                                             needs_layout_passes=False),
        scratch_types=[
            pltpu.VMEM((NODE_CHUNK, D_N), jnp.float32),
            pltpu.VMEM((NODE_CHUNK,), jnp.int32),
            pltpu.VMEM((LANES, D_N), jnp.float32),
            pltpu.VMEM((LANES,), jnp.int32),
            pltpu.VMEM((G, D_N), jnp.float32),
            pltpu.SemaphoreType.DMA,
            pltpu.SemaphoreType.DMA,
        ],
    )
    def k(nf_hbm, nid_hbm, pn_hbm, nbuf, nidv, ntbuf, ntidv, acc_n,
          sem_n, sem_ni):
        wid = lax.axis_index("c") * NS + lax.axis_index("s")
        zero = jnp.zeros((LANES,), jnp.float32)

        nbase = wid * NODE_CHUNK
        h_ni = pltpu.async_copy(nid_hbm.at[pl.ds(nbase, NODE_CHUNK)], nidv,
                                sem_ni)
        h_n = pltpu.async_copy(nf_hbm.at[pl.ds(nbase, NODE_CHUNK)], nbuf,
                               sem_n)

        def zbody(g, carry):
            for j in range(NJ):
                acc_n[g, pl.ds(j * LANES, LANES)] = zero
            return carry
        lax.fori_loop(0, G, zbody, 0)

        h_ni.wait()
        h_n.wait()

        def nbody(grp, carry):
            prev, run = carry[0], carry[1:]
            i0 = grp * LANES
            gids = nidv[pl.ds(i0, LANES)]
            last = gids[LANES - 1]
            rows = [[nbuf[i0 + l, pl.ds(j * LANES, LANES)] for j in range(NJ)]
                    for l in range(LANES)]

            def same_fn():
                s = list(run)
                for l in range(LANES):
                    s = [s[j] + rows[l][j] for j in range(NJ)]
                return (prev,) + tuple(s)

            def diff_fn():
                for j in range(NJ):
                    plsc.addupdate(acc_n.at[prev, pl.ds(j * LANES, LANES)],
                                   run[j])
                for l in range(LANES):
                    g = gids[l]
                    for j in range(NJ):
                        plsc.addupdate(acc_n.at[g, pl.ds(j * LANES, LANES)],
                                       rows[l][j])
                return (last,) + (zero,) * NJ

            return lax.cond(last == prev, same_fn, diff_fn)

        ncarry = lax.fori_loop(0, NODE_CHUNK // LANES, nbody,
                               (jnp.int32(0),) + (zero,) * NJ)
        for j in range(NJ):
            plsc.addupdate(acc_n.at[ncarry[0], pl.ds(j * LANES, LANES)],
                           ncarry[1 + j])

        @pl.when(wid < NODE_TAIL_GROUPS)
        def _tail():
            tb = NW * NODE_CHUNK + wid * LANES
            pltpu.sync_copy(nid_hbm.at[pl.ds(tb, LANES)], ntidv)
            pltpu.sync_copy(nf_hbm.at[pl.ds(tb, LANES)], ntbuf)
            gids = ntidv[...]
            for l in range(LANES):
                g = gids[l]
                for j in range(NJ):
                    plsc.addupdate(acc_n.at[g, pl.ds(j * LANES, LANES)],
                                   ntbuf[l, pl.ds(j * LANES, LANES)])

        pltpu.sync_copy(acc_n, pn_hbm.at[wid])

    return k(node_feat, node_ids)


def _sc_edge_partials(edge_feat_t, edge_ids):
    mesh = plsc.VectorSubcoreMesh(core_axis_name="c", subcore_axis_name="s")

    @functools.partial(
        pl.kernel,
        out_type=jax.ShapeDtypeStruct((NW, D_E, G), jnp.float32),
        mesh=mesh,
        compiler_params=pltpu.CompilerParams(use_tc_tiling_on_sc=False,
                                             needs_layout_passes=False),
        scratch_types=[
            pltpu.VMEM((D_E, E_CHUNK), jnp.float32),
            pltpu.VMEM((D_E, E_CHUNK), jnp.float32),
            pltpu.VMEM((EDGE_PER_W,), jnp.int32),
            pltpu.VMEM((D_E, G), jnp.float32),
            pltpu.SemaphoreType.DMA,
            pltpu.SemaphoreType.DMA,
            pltpu.SemaphoreType.DMA,
        ],
    )
    def k(eft_hbm, eid_hbm, pet_hbm, ebuf0, ebuf1, eidv, acc_et,
          sem_i, sem_e0, sem_e1):
        wid = lax.axis_index("c") * NS + lax.axis_index("s")
        zero = jnp.zeros((LANES,), jnp.float32)
        ebufs = (ebuf0, ebuf1)
        esems = (sem_e0, sem_e1)

        ebase = wid * EDGE_PER_W
        h_id = pltpu.async_copy(eid_hbm.at[pl.ds(ebase, EDGE_PER_W)], eidv,
                                sem_i)
        eh = [None] * N_ECHUNKS
        eh[0] = pltpu.async_copy(eft_hbm.at[:, pl.ds(ebase, E_CHUNK)],
                                 ebufs[0], sem_e0)

        for f in range(D_E):
            for g4 in range(G // LANES):
                acc_et[f, pl.ds(g4 * LANES, LANES)] = zero

        h_id.wait()
        prev = jnp.int32(0)
        run = (zero,) * D_E
        for c in range(N_ECHUNKS):
            if c + 1 < N_ECHUNKS:
                nxt = (c + 1) % 2
                eh[c + 1] = pltpu.async_copy(
                    eft_hbm.at[:, pl.ds(ebase + (c + 1) * E_CHUNK, E_CHUNK)],
                    ebufs[nxt], esems[nxt])
            eh[c].wait()
            buf = ebufs[c % 2]

            off = 0
            for sub in E_SUBS:
                gl = eidv[pl.ds(c * E_CHUNK + off + sub - LANES, LANES)]
                u_last = gl[LANES - 1]

                def pure_fn(off=off, sub=sub, buf=buf, prev=prev, run=run):
                    def pbody(grp, rc, off=off, buf=buf):
                        i0 = off + grp * LANES
                        return tuple(rc[f] + buf[f, pl.ds(i0, LANES)]
                                     for f in range(D_E))
                    return ((prev,)
                            + lax.fori_loop(0, sub // LANES, pbody, run))

                def fb_fn(off=off, sub=sub, buf=buf, prev=prev, run=run, c=c):
                    def ebody(grp, ecarry, off=off, buf=buf, c=c):
                        pv_, rn = ecarry[0], ecarry[1:]
                        i0 = off + grp * LANES
                        gids = eidv[pl.ds(c * E_CHUNK + i0, LANES)]
                        last = gids[LANES - 1]
                        v = [buf[f, pl.ds(i0, LANES)] for f in range(D_E)]

                        def same_fn():
                            return (pv_,) + tuple(rn[f] + v[f]
                                                  for f in range(D_E))

                        def diff_fn():
                            pv = jnp.full((LANES,), pv_, jnp.int32)
                            for f in range(D_E):
                                plsc.addupdate_scatter(acc_et.at[f], [pv],
                                                       rn[f])
                            for f in range(D_E):
                                plsc.addupdate_scatter(acc_et.at[f], [gids],
                                                       v[f])
                            return (last,) + (zero,) * D_E

                        return lax.cond(last == pv_, same_fn, diff_fn)

                    return lax.fori_loop(0, sub // LANES, ebody,
                                         (prev,) + run)

                res = lax.cond(u_last == prev, pure_fn, fb_fn)
                prev, run = res[0], res[1:]
                off += sub
        carry = (prev,) + run

        pv = jnp.full((LANES,), carry[0], jnp.int32)
        for f in range(D_E):
            plsc.addupdate_scatter(acc_et.at[f], [pv], carry[1 + f])

        pltpu.sync_copy(acc_et, pet_hbm.at[wid])

    return k(edge_feat_t, edge_ids)


def _combine_body(pn_ref, pet_ref, out_ref):
    out_ref[:, :D_N] = jnp.sum(pn_ref[...], axis=0)
    es_t = jnp.sum(pet_ref[...], axis=0)
    out_ref[:, D_N:] = es_t.T


def kernel(node_feat, node_graph_ids, edge_feat, edge_graph_ids, num_graphs):
    del num_graphs
    pn = _sc_node_partials(node_feat, node_graph_ids.astype(jnp.int32))
    pn, eids = lax.optimization_barrier(
        (pn, edge_graph_ids.astype(jnp.int32)))
    pet = _sc_edge_partials(edge_feat.T, eids)
    return pl.pallas_call(
        _combine_body,
        out_shape=jax.ShapeDtypeStruct((G, D_N + D_E), jnp.float32),
    )(pn, pet)
